# Initial kernel scaffold; baseline (speedup 1.0000x reference)
import jax
import jax.numpy as jnp
from jax.experimental import pallas as pl


def kernel(x, weight):
    raise NotImplementedError("write your pallas kernel here")


# speedup vs baseline: 1.6841x; 1.6841x over previous
import functools

import jax
import jax.numpy as jnp
from jax import lax
from jax.experimental import pallas as pl
from jax.experimental.pallas import tpu as pltpu
from jax.experimental.pallas import tpu_sc as plsc

DIM = 64
BATCH, SEQ = 16384, 50
TOTAL = BATCH * SEQ
NC, NS = 2, 16
NW = NC * NS
PER_W = TOTAL // NW
CHUNK = 128
N_CHUNKS = PER_W // CHUNK

_mesh = plsc.VectorSubcoreMesh(core_axis_name="c", subcore_axis_name="s")


@functools.partial(
    pl.kernel,
    mesh=_mesh,
    compiler_params=pltpu.CompilerParams(use_tc_tiling_on_sc=False),
    out_type=jax.ShapeDtypeStruct((TOTAL, DIM), jnp.float32),
    scratch_types=[
        pltpu.VMEM((N_CHUNKS, CHUNK), jnp.int32),
        pltpu.VMEM((CHUNK, DIM), jnp.float32),
        pltpu.SemaphoreType.DMA,
    ],
)
def _embed_sc(x_hbm, w_hbm, out_hbm, idx_v, rows, gsem):
    wid = lax.axis_index("s") * NC + lax.axis_index("c")
    base = wid * PER_W
    pltpu.sync_copy(x_hbm.at[wid], idx_v)

    def body(g, carry):
        pltpu.async_copy(w_hbm.at[idx_v.at[g]], rows, gsem).wait()
        pltpu.sync_copy(rows, out_hbm.at[pl.ds(base + g * CHUNK, CHUNK)])
        return carry

    lax.fori_loop(0, N_CHUNKS, body, 0)


def kernel(x, weight):
    xf = x.reshape(NW, N_CHUNKS, CHUNK)
    out = _embed_sc(xf, weight)
    return out.reshape(BATCH, SEQ, DIM)


# speedup vs baseline: 1.8644x; 1.1070x over previous
import functools

import jax
import jax.numpy as jnp
from jax import lax
from jax.experimental import pallas as pl
from jax.experimental.pallas import tpu as pltpu
from jax.experimental.pallas import tpu_sc as plsc

DIM = 64
BATCH, SEQ = 16384, 50
TOTAL = BATCH * SEQ
NC, NS = 2, 16
NW = NC * NS
PER_W = TOTAL // NW
CHUNK = 128
N_CHUNKS = PER_W // CHUNK
GPC = 4
GROUP = CHUNK * GPC
PAIRS = N_CHUNKS // (2 * GPC)

_mesh = plsc.VectorSubcoreMesh(core_axis_name="c", subcore_axis_name="s")


@functools.partial(
    pl.kernel,
    mesh=_mesh,
    compiler_params=pltpu.CompilerParams(use_tc_tiling_on_sc=False),
    out_type=jax.ShapeDtypeStruct((TOTAL, DIM), jnp.float32),
    scratch_types=[
        pltpu.VMEM((N_CHUNKS, CHUNK), jnp.int32),
        pltpu.VMEM((GROUP, DIM), jnp.float32),
        pltpu.VMEM((GROUP, DIM), jnp.float32),
        pltpu.SemaphoreType.DMA,
        pltpu.SemaphoreType.DMA,
        pltpu.SemaphoreType.DMA,
        pltpu.SemaphoreType.DMA,
    ],
)
def _embed_sc(x_hbm, w_hbm, out_hbm, idx_v, buf_a, buf_b, gsem_a, gsem_b,
              wsem_a, wsem_b):
    wid = lax.axis_index("s") * NC + lax.axis_index("c")
    base = wid * PER_W
    pltpu.sync_copy(x_hbm.at[wid], idx_v)

    def start_gathers(group, buf, sem):
        for b in range(GPC):
            pltpu.async_copy(w_hbm.at[idx_v.at[group * GPC + b]],
                             buf.at[pl.ds(b * CHUNK, CHUNK)], sem)

    def wait_gathers(buf, sem):
        for b in range(GPC):
            pltpu.make_async_copy(w_hbm.at[idx_v.at[0]],
                                  buf.at[pl.ds(b * CHUNK, CHUNK)], sem).wait()

    def start_write(group, buf, sem):
        pltpu.async_copy(buf, out_hbm.at[pl.ds(base + group * GROUP, GROUP)], sem)

    def wait_write(buf, sem):
        pltpu.make_async_copy(buf, out_hbm.at[pl.ds(base, GROUP)], sem).wait()

    start_gathers(0, buf_a, gsem_a)

    def body(k, carry):
        wait_gathers(buf_a, gsem_a)
        start_write(2 * k, buf_a, wsem_a)

        @pl.when(k > 0)
        def _():
            wait_write(buf_b, wsem_b)

        start_gathers(2 * k + 1, buf_b, gsem_b)

        wait_write(buf_a, wsem_a)

        @pl.when(k < PAIRS - 1)
        def _():
            start_gathers(2 * k + 2, buf_a, gsem_a)

        wait_gathers(buf_b, gsem_b)
        start_write(2 * k + 1, buf_b, wsem_b)
        return carry

    lax.fori_loop(0, PAIRS, body, 0)
    wait_write(buf_b, wsem_b)


def kernel(x, weight):
    xf = x.reshape(NW, N_CHUNKS, CHUNK)
    out = _embed_sc(xf, weight)
    return out.reshape(BATCH, SEQ, DIM)
